# baseline (speedup 1.0000x reference)
import dataclasses

import jax
import jax.numpy as jnp
from jax import lax
from jax.experimental import pallas as pl
from jax.experimental.pallas import tpu as pltpu
from jax.experimental.pallas import tpu_sc as plsc

C = 1000
B = 16384
BB = 2048
NB = B // BB

CP = 1024
FLAT = C * CP

NSUB = 16
NCORE = 2
HALF = B // NCORE
SHIFT = 16
BINS = 1 << SHIFT
BINS_15 = FLAT - 15 * BINS


def _argmax_body(pred_ref, tgt_ref, out_ref):
    x = pred_ref[...]
    mx = jnp.max(x, axis=0, keepdims=True)
    row = jax.lax.broadcasted_iota(jnp.int32, x.shape, 0)
    p = jnp.min(jnp.where(x == mx, row, C), axis=0)
    t = tgt_ref[0, 0, :]
    out_ref[0, 0, :] = (
        ((t >> 3) << 13) + ((p >> 7) << 10) + ((t & 7) << 7) + (p & 127)
    )


def _flat_indices(prediction, target):
    pred_t = prediction.T
    tgt3 = target.reshape(NB, 1, BB)
    out = pl.pallas_call(
        _argmax_body,
        grid=(NB,),
        in_specs=[
            pl.BlockSpec((C, BB), lambda i: (0, i)),
            pl.BlockSpec((1, 1, BB), lambda i: (i, 0, 0)),
        ],
        out_specs=pl.BlockSpec((1, 1, BB), lambda i: (i, 0, 0)),
        out_shape=jax.ShapeDtypeStruct((NB, 1, BB), jnp.int32),
        compiler_params=pltpu.CompilerParams(
            dimension_semantics=("parallel",),
        ),
    )(pred_t, tgt3)
    return out.reshape(B)


def _sc_histogram(flat_idx):
    mesh = plsc.VectorSubcoreMesh(core_axis_name="c", subcore_axis_name="s")
    cp = pltpu.CompilerParams()
    if "needs_layout_passes" in pltpu.CompilerParams.__dataclass_fields__:
        cp = dataclasses.replace(cp, needs_layout_passes=False)

    @pl.kernel(
        compiler_params=cp,
        out_type=(
            jax.ShapeDtypeStruct((FLAT,), jnp.float32),
            jax.ShapeDtypeStruct((FLAT,), jnp.float32),
        ),
        mesh=mesh,
        scratch_types=[
            pltpu.VMEM((HALF,), jnp.int32),
            pltpu.VMEM((BINS,), jnp.float32),
        ],
    )
    def hist_kernel(idx_hbm, out0_hbm, out1_hbm, idx_v, bins_v):
        cid = lax.axis_index("c")
        sid = lax.axis_index("s")

        zeros = jnp.zeros((16,), jnp.float32)

        @pl.loop(0, BINS, step=64)
        def _(i):
            for k in range(4):
                bins_v.at[pl.ds(i + 16 * k, 16)][...] = zeros

        pltpu.sync_copy(idx_hbm.at[pl.ds(cid * HALF, HALF)], idx_v)

        ones = jnp.full((16,), 1.0, jnp.float32)

        @pl.loop(0, HALF, step=64)
        def _(i):
            for k in range(4):
                v = idx_v.at[pl.ds(i + 16 * k, 16)][...]
                m = (v >> SHIFT) == sid
                local = v & (BINS - 1)
                plsc.addupdate_scatter(bins_v, [local], ones, mask=m)

        for c, out_hbm in ((0, out0_hbm), (1, out1_hbm)):
            @pl.when(jnp.logical_and(cid == c, sid < 15))
            def _():
                pltpu.sync_copy(bins_v, out_hbm.at[pl.ds(sid * BINS, BINS)])

            @pl.when(jnp.logical_and(cid == c, sid == 15))
            def _():
                pltpu.sync_copy(
                    bins_v.at[pl.ds(0, BINS_15)],
                    out_hbm.at[pl.ds(15 * BINS, BINS_15)],
                )

    return hist_kernel(flat_idx)


MK = 5
MROWS = 8 * MK
MIN_ROWS = 64 * MK


def _merge_body(a_ref, b_ref, out_ref):
    s = a_ref[...] + b_ref[...]
    y = s.reshape(MK, 8, 8, 128).swapaxes(1, 2).reshape(MROWS, CP)
    out_ref[...] = y[:, :C]


def _merge(part0, part1):
    a = part0.reshape(FLAT // 128, 128)
    b = part1.reshape(FLAT // 128, 128)
    return pl.pallas_call(
        _merge_body,
        grid=(C // MROWS,),
        in_specs=[
            pl.BlockSpec((MIN_ROWS, 128), lambda i: (i, 0)),
            pl.BlockSpec((MIN_ROWS, 128), lambda i: (i, 0)),
        ],
        out_specs=pl.BlockSpec((MROWS, C), lambda i: (i, 0)),
        out_shape=jax.ShapeDtypeStruct((C, C), jnp.float32),
        compiler_params=pltpu.CompilerParams(
            dimension_semantics=("parallel",),
        ),
    )(a, b)


def kernel(prediction, target):
    flat_idx = _flat_indices(prediction, target)
    part0, part1 = _sc_histogram(flat_idx)
    return _merge(part0, part1)


# speedup vs baseline: 1.1232x; 1.1232x over previous
import dataclasses

import jax
import jax.numpy as jnp
from jax import lax
from jax.experimental import pallas as pl
from jax.experimental.pallas import tpu as pltpu
from jax.experimental.pallas import tpu_sc as plsc

C = 1000
B = 16384
BB = 2048
NB = B // BB

NSUB = 16
NCORE = 2
NTILE = NCORE * NSUB
SHIFT = 15
BINS = 1 << SHIFT
BINS_30 = C * C - 30 * BINS


def _argmax_body(pred_ref, tgt_ref, out_ref):
    x = pred_ref[...]
    mx = jnp.max(x, axis=0, keepdims=True)
    row = jax.lax.broadcasted_iota(jnp.int32, x.shape, 0)
    p = jnp.min(jnp.where(x == mx, row, C), axis=0)
    out_ref[0, 0, :] = tgt_ref[0, 0, :] * C + p


def _flat_indices(prediction, target):
    pred_t = prediction.T
    tgt3 = target.reshape(NB, 1, BB)
    out = pl.pallas_call(
        _argmax_body,
        grid=(NB,),
        in_specs=[
            pl.BlockSpec((C, BB), lambda i: (0, i)),
            pl.BlockSpec((1, 1, BB), lambda i: (i, 0, 0)),
        ],
        out_specs=pl.BlockSpec((1, 1, BB), lambda i: (i, 0, 0)),
        out_shape=jax.ShapeDtypeStruct((NB, 1, BB), jnp.int32),
        compiler_params=pltpu.CompilerParams(
            dimension_semantics=("parallel",),
        ),
    )(pred_t, tgt3)
    return out.reshape(B)


def _sc_histogram(flat_idx):
    mesh = plsc.VectorSubcoreMesh(core_axis_name="c", subcore_axis_name="s")
    cp = pltpu.CompilerParams()
    if "needs_layout_passes" in pltpu.CompilerParams.__dataclass_fields__:
        cp = dataclasses.replace(cp, needs_layout_passes=False)

    @pl.kernel(
        compiler_params=cp,
        out_type=jax.ShapeDtypeStruct((C * C,), jnp.float32),
        mesh=mesh,
        scratch_types=[
            pltpu.VMEM((B,), jnp.int32),
            pltpu.VMEM((BINS,), jnp.float32),
        ],
    )
    def hist_kernel(idx_hbm, out_hbm, idx_v, bins_v):
        cid = lax.axis_index("c")
        sid = lax.axis_index("s")
        wid = cid * NSUB + sid

        zeros = jnp.zeros((16,), jnp.float32)

        @pl.loop(0, BINS, step=64)
        def _(i):
            for k in range(4):
                bins_v.at[pl.ds(i + 16 * k, 16)][...] = zeros

        pltpu.sync_copy(idx_hbm, idx_v)

        ones = jnp.full((16,), 1.0, jnp.float32)

        @pl.loop(0, B, step=64)
        def _(i):
            for k in range(4):
                v = idx_v.at[pl.ds(i + 16 * k, 16)][...]
                m = (v >> SHIFT) == wid
                local = v & (BINS - 1)
                plsc.addupdate_scatter(bins_v, [local], ones, mask=m)

        @pl.when(wid < 30)
        def _():
            pltpu.sync_copy(
                bins_v, out_hbm.at[pl.ds(wid * BINS, BINS)]
            )

        @pl.when(wid == 30)
        def _():
            pltpu.sync_copy(
                bins_v.at[pl.ds(0, BINS_30)],
                out_hbm.at[pl.ds(30 * BINS, BINS_30)],
            )

    return hist_kernel(flat_idx)


def kernel(prediction, target):
    flat_idx = _flat_indices(prediction, target)
    cm_flat = _sc_histogram(flat_idx)
    return cm_flat.reshape(C, C)
